# baseline (speedup 1.0000x reference)
import functools

import jax
import jax.numpy as jnp
from jax import lax
from jax.experimental import pallas as pl
from jax.experimental.pallas import tpu as pltpu
from jax.experimental.pallas import tpu_sc as plsc

N_NODES = 10000
N_PAD = 10112
E_EDGES = 320000
E_PAD = 327680
F_IN = 128
H_DIM = 16
C_DIM = 100

NW = 32
EW = E_PAD // NW
CHUNK = 1024
NCHUNK = EW // CHUNK
SUB = CHUNK // 128
ROWS_PER_TILE = N_PAD // 16

ROWS_REAL = E_EDGES // 128
ROWS_W = EW // 128
NCH = ROWS_W // SUB

import numpy as _np
_N_PAD_ROWS = (E_PAD - E_EDGES) // 128
_SRC_PAD = (_np.arange(E_PAD - E_EDGES, dtype=_np.int32) * 7919 % N_NODES
            ).reshape(_N_PAD_ROWS, 128)
_DST_PAD = (N_NODES + _np.arange(E_PAD - E_EDGES, dtype=_np.int32)
            % (N_PAD - N_NODES)).reshape(_N_PAD_ROWS, 128)

_mesh = plsc.VectorSubcoreMesh(core_axis_name="c", subcore_axis_name="s")
_sc_params = pltpu.CompilerParams(use_tc_tiling_on_sc=False)


def _zero_rows(buf, nrows):
    def body(i, carry):
        buf[i, :] = jnp.zeros((16,), jnp.float32)
        return carry
    lax.fori_loop(0, nrows, body, 0)


def _fill_ones(buf, nrows):
    def body(i, carry):
        buf[i, :] = jnp.ones((16,), jnp.float32)
        return carry
    lax.fori_loop(0, nrows, body, 0)


@functools.partial(
    pl.kernel,
    mesh=_mesh,
    out_type=jax.ShapeDtypeStruct((2, N_PAD, H_DIM), jnp.float32),
    scratch_types=[
        pltpu.VMEM((ROWS_W, 128), jnp.int32),
        pltpu.VMEM((ROWS_W, 128), jnp.int32),
        pltpu.VMEM((2, CHUNK, H_DIM), jnp.float32),
        pltpu.VMEM_SHARED((N_PAD, H_DIM), jnp.float32),
        pltpu.SemaphoreType.DMA,
        pltpu.SemaphoreType.DMA,
        pltpu.SemaphoreType.DMA,
        pltpu.SemaphoreType.DMA,
    ],
    compiler_params=_sc_params,
)
def _seg_sum(table, src2d, dst2d, src_pad, dst_pad, out, src_v, dst_v, rows_v,
             acc_sh, gsem0, gsem1, ssem0, ssem1):
    cid = lax.axis_index("c")
    sid = lax.axis_index("s")
    wid = cid * 16 + sid
    gsem = (gsem0, gsem1)
    ssem = (ssem0, ssem1)
    _zero_rows(rows_v.at[0], ROWS_PER_TILE)
    pltpu.sync_copy(rows_v.at[0, pl.ds(0, ROWS_PER_TILE)],
                    acc_sh.at[pl.ds(sid * ROWS_PER_TILE, ROWS_PER_TILE)])

    @pl.when(wid < NW - 1)
    def _():
        pltpu.sync_copy(src2d.at[pl.ds(wid * ROWS_W, ROWS_W)], src_v)
        pltpu.sync_copy(dst2d.at[pl.ds(wid * ROWS_W, ROWS_W)], dst_v)

    @pl.when(wid == NW - 1)
    def _():
        tail = ROWS_REAL - (NW - 1) * ROWS_W
        pltpu.sync_copy(src2d.at[pl.ds((NW - 1) * ROWS_W, tail)],
                        src_v.at[pl.ds(0, tail)])
        pltpu.sync_copy(dst2d.at[pl.ds((NW - 1) * ROWS_W, tail)],
                        dst_v.at[pl.ds(0, tail)])
        pltpu.sync_copy(src_pad, src_v.at[pl.ds(tail, _N_PAD_ROWS)])
        pltpu.sync_copy(dst_pad, dst_v.at[pl.ds(tail, _N_PAD_ROWS)])

    plsc.subcore_barrier()

    def pipeline(n_chunks):
        def fire_gathers(k, b):
            return [
                pltpu.async_copy(table.at[src_v.at[k * SUB + j]],
                                 rows_v.at[b, pl.ds(j * 128, 128)], gsem[b])
                for j in range(SUB)
            ]

        g_pend = {0: fire_gathers(0, 0)}
        s_pend = {}
        for k in range(n_chunks):
            b = k % 2
            nb = 1 - b
            if k >= 1:
                for c in s_pend.pop(k - 1):
                    c.wait()
            if k + 1 < n_chunks:
                g_pend[k + 1] = fire_gathers(k + 1, nb)
            for c in g_pend.pop(k):
                c.wait()
            s_pend[k] = [
                pltpu.async_copy(rows_v.at[b, pl.ds(j * 128, 128)],
                                 acc_sh.at[dst_v.at[k * SUB + j]], ssem[b],
                                 add=True)
                for j in range(SUB)
            ]
        for c in s_pend.pop(n_chunks - 1):
            c.wait()

    pipeline(NCH)
    plsc.subcore_barrier()
    pltpu.sync_copy(acc_sh.at[pl.ds(sid * ROWS_PER_TILE, ROWS_PER_TILE)],
                    out.at[cid, pl.ds(sid * ROWS_PER_TILE, ROWS_PER_TILE)])


@functools.partial(
    pl.kernel,
    mesh=_mesh,
    out_type=jax.ShapeDtypeStruct((2, N_PAD, H_DIM), jnp.float32),
    scratch_types=[
        pltpu.VMEM((ROWS_W, 128), jnp.int32),
        pltpu.VMEM((128, H_DIM), jnp.float32),
        pltpu.VMEM_SHARED((N_PAD, H_DIM), jnp.float32),
        pltpu.SemaphoreType.DMA,
    ],
    compiler_params=_sc_params,
)
def _deg_count(dst2d, dst_pad, out, dst_v, ones_v, acc_sh, sem):
    cid = lax.axis_index("c")
    sid = lax.axis_index("s")
    wid = cid * 16 + sid
    _zero_rows(ones_v, 128)
    def zslab(i, carry):
        pltpu.sync_copy(ones_v.at[pl.ds(0, 8)],
                        acc_sh.at[pl.ds(sid * ROWS_PER_TILE + i * 8, 8)])
        return carry
    lax.fori_loop(0, ROWS_PER_TILE // 8, zslab, 0)
    _fill_ones(ones_v, 128)

    @pl.when(wid < NW - 1)
    def _():
        pltpu.sync_copy(dst2d.at[pl.ds(wid * ROWS_W, ROWS_W)], dst_v)

    @pl.when(wid == NW - 1)
    def _():
        tail = ROWS_REAL - (NW - 1) * ROWS_W
        pltpu.sync_copy(dst2d.at[pl.ds((NW - 1) * ROWS_W, tail)],
                        dst_v.at[pl.ds(0, tail)])
        pltpu.sync_copy(dst_pad, dst_v.at[pl.ds(tail, _N_PAD_ROWS)])

    plsc.subcore_barrier()

    pend = []
    for j in range(ROWS_W):
        pend.append(pltpu.async_copy(ones_v, acc_sh.at[dst_v.at[j]], sem,
                                     add=True))
        if len(pend) > 16:
            pend.pop(0).wait()
    for c in pend:
        c.wait()
    plsc.subcore_barrier()
    pltpu.sync_copy(acc_sh.at[pl.ds(sid * ROWS_PER_TILE, ROWS_PER_TILE)],
                    out.at[cid, pl.ds(sid * ROWS_PER_TILE, ROWS_PER_TILE)])


_ROWS_BLK = 1000
_NBLK = N_NODES // _ROWS_BLK


def _tc_a_body(x_ref, w1_ref, degp_ref, s1_ref, norm_ref):
    cnt = degp_ref[0, :, 0:1] + degp_ref[1, :, 0:1] + 1.0
    nrm = lax.rsqrt(jnp.maximum(cnt, 1.0))
    hw = jnp.dot(x_ref[...], w1_ref[...], preferred_element_type=jnp.float32)
    s1_ref[...] = hw * nrm
    norm_ref[...] = nrm


def _tc_b_body(g1p_ref, s1_ref, norm_ref, b1_ref, s2_ref):
    nrm = norm_ref[...]
    h1 = jnp.maximum((g1p_ref[0] + g1p_ref[1] + s1_ref[...]) * nrm
                     + b1_ref[...], 0.0)
    s2_ref[...] = h1 * nrm


def _tc_c_body(g2p_ref, s2_ref, norm_ref, w2_ref, b2_ref, out_ref):
    agg = (g2p_ref[0] + g2p_ref[1] + s2_ref[...]) * norm_ref[...]
    z = jnp.dot(agg, w2_ref[...], preferred_element_type=jnp.float32) + b2_ref[...]
    m = jnp.max(z, axis=1, keepdims=True)
    lse = m + jnp.log(jnp.sum(jnp.exp(z - m), axis=1, keepdims=True))
    out_ref[...] = z - lse


_tc_a = pl.pallas_call(
    _tc_a_body,
    grid=(_NBLK,),
    in_specs=[
        pl.BlockSpec((_ROWS_BLK, F_IN), lambda i: (i, 0)),
        pl.BlockSpec((F_IN, H_DIM), lambda i: (0, 0)),
        pl.BlockSpec((2, _ROWS_BLK, H_DIM), lambda i: (0, i, 0)),
    ],
    out_specs=[
        pl.BlockSpec((_ROWS_BLK, H_DIM), lambda i: (i, 0)),
        pl.BlockSpec((_ROWS_BLK, 1), lambda i: (i, 0)),
    ],
    out_shape=[
        jax.ShapeDtypeStruct((N_NODES, H_DIM), jnp.float32),
        jax.ShapeDtypeStruct((N_NODES, 1), jnp.float32),
    ],
)

_tc_b = pl.pallas_call(
    _tc_b_body,
    grid=(_NBLK,),
    in_specs=[
        pl.BlockSpec((2, _ROWS_BLK, H_DIM), lambda i: (0, i, 0)),
        pl.BlockSpec((_ROWS_BLK, H_DIM), lambda i: (i, 0)),
        pl.BlockSpec((_ROWS_BLK, 1), lambda i: (i, 0)),
        pl.BlockSpec((1, H_DIM), lambda i: (0, 0)),
    ],
    out_specs=pl.BlockSpec((_ROWS_BLK, H_DIM), lambda i: (i, 0)),
    out_shape=jax.ShapeDtypeStruct((N_NODES, H_DIM), jnp.float32),
)

_tc_c = pl.pallas_call(
    _tc_c_body,
    grid=(_NBLK,),
    in_specs=[
        pl.BlockSpec((2, _ROWS_BLK, H_DIM), lambda i: (0, i, 0)),
        pl.BlockSpec((_ROWS_BLK, H_DIM), lambda i: (i, 0)),
        pl.BlockSpec((_ROWS_BLK, 1), lambda i: (i, 0)),
        pl.BlockSpec((H_DIM, C_DIM), lambda i: (0, 0)),
        pl.BlockSpec((1, C_DIM), lambda i: (0, 0)),
    ],
    out_specs=pl.BlockSpec((_ROWS_BLK, C_DIM), lambda i: (i, 0)),
    out_shape=jax.ShapeDtypeStruct((N_NODES, C_DIM), jnp.float32),
)


def kernel(x, edge_index, W1, b1, W2, b2):
    src2d = edge_index[0].reshape(ROWS_REAL, 128)
    dst2d = edge_index[1].reshape(ROWS_REAL, 128)
    src_pad = jnp.asarray(_SRC_PAD)
    dst_pad = jnp.asarray(_DST_PAD)

    degp = _deg_count(dst2d, dst_pad)
    s1, norm = _tc_a(x, W1, degp)
    g1p = _seg_sum(s1, src2d, dst2d, src_pad, dst_pad)
    s2 = _tc_b(g1p, s1, norm, b1.reshape(1, H_DIM))
    g2p = _seg_sum(s2, src2d, dst2d, src_pad, dst_pad)
    return _tc_c(g2p, s2, norm, W2, b2.reshape(1, C_DIM))


# speedup vs baseline: 1.0167x; 1.0167x over previous
import functools

import jax
import jax.numpy as jnp
from jax import lax
from jax.experimental import pallas as pl
from jax.experimental.pallas import tpu as pltpu
from jax.experimental.pallas import tpu_sc as plsc

N_NODES = 10000
N_PAD = 10112
E_EDGES = 320000
E_PAD = 327680
F_IN = 128
H_DIM = 16
C_DIM = 100

NW = 32
EW = E_PAD // NW
CHUNK = 1024
NCHUNK = EW // CHUNK
SUB = CHUNK // 128
ROWS_PER_TILE = N_PAD // 16

ROWS_TOTAL = E_PAD // 128
CH0 = 10
CH1 = 10
ROWS_W0 = CH0 * SUB
ROWS_W1 = CH1 * SUB
CORE1_BASE = 16 * ROWS_W0

_mesh = plsc.VectorSubcoreMesh(core_axis_name="c", subcore_axis_name="s")
_sc_params = pltpu.CompilerParams(use_tc_tiling_on_sc=False)


def _zero_rows(buf, nrows):
    def body(i, carry):
        buf[i, :] = jnp.zeros((16,), jnp.float32)
        return carry
    lax.fori_loop(0, nrows, body, 0)


def _fill_ones(buf, nrows):
    def body(i, carry):
        buf[i, :] = jnp.ones((16,), jnp.float32)
        return carry
    lax.fori_loop(0, nrows, body, 0)


@functools.partial(
    pl.kernel,
    mesh=_mesh,
    out_type=jax.ShapeDtypeStruct((2, N_PAD, H_DIM), jnp.float32),
    scratch_types=[
        pltpu.VMEM((ROWS_W0, 128), jnp.int32),
        pltpu.VMEM((ROWS_W0, 128), jnp.int32),
        pltpu.VMEM((2, CHUNK, H_DIM), jnp.float32),
        pltpu.VMEM_SHARED((N_PAD, H_DIM), jnp.float32),
        pltpu.SemaphoreType.DMA,
        pltpu.SemaphoreType.DMA,
        pltpu.SemaphoreType.DMA,
        pltpu.SemaphoreType.DMA,
    ],
    compiler_params=_sc_params,
)
def _seg_sum(table, src2d, dst2d, out, src_v, dst_v, rows_v,
             acc_sh, gsem0, gsem1, ssem0, ssem1):
    cid = lax.axis_index("c")
    sid = lax.axis_index("s")
    gsem = (gsem0, gsem1)
    ssem = (ssem0, ssem1)
    _zero_rows(rows_v.at[0], ROWS_PER_TILE)
    pltpu.sync_copy(rows_v.at[0, pl.ds(0, ROWS_PER_TILE)],
                    acc_sh.at[pl.ds(sid * ROWS_PER_TILE, ROWS_PER_TILE)])
    plsc.subcore_barrier()

    def pipeline(base_row, n_chunks):
        nrows = n_chunks * SUB
        pltpu.sync_copy(src2d.at[pl.ds(base_row, nrows)],
                        src_v.at[pl.ds(0, nrows)])
        pltpu.sync_copy(dst2d.at[pl.ds(base_row, nrows)],
                        dst_v.at[pl.ds(0, nrows)])

        def fire_gathers(k, b):
            return [
                pltpu.async_copy(table.at[src_v.at[k * SUB + j]],
                                 rows_v.at[b, pl.ds(j * 128, 128)], gsem[b])
                for j in range(SUB)
            ]

        g_pend = {0: fire_gathers(0, 0)}
        s_pend = {}
        for k in range(n_chunks):
            b = k % 2
            nb = 1 - b
            if k >= 1:
                for c in s_pend.pop(k - 1):
                    c.wait()
            if k + 1 < n_chunks:
                g_pend[k + 1] = fire_gathers(k + 1, nb)
            for c in g_pend.pop(k):
                c.wait()
            s_pend[k] = [
                pltpu.async_copy(rows_v.at[b, pl.ds(j * 128, 128)],
                                 acc_sh.at[dst_v.at[k * SUB + j]], ssem[b],
                                 add=True)
                for j in range(SUB)
            ]
        for c in s_pend.pop(n_chunks - 1):
            c.wait()

    @pl.when(cid == 0)
    def _():
        pipeline(sid * ROWS_W0, CH0)

    @pl.when(cid == 1)
    def _():
        pipeline(CORE1_BASE + sid * ROWS_W1, CH1)

    plsc.subcore_barrier()
    pltpu.sync_copy(acc_sh.at[pl.ds(sid * ROWS_PER_TILE, ROWS_PER_TILE)],
                    out.at[cid, pl.ds(sid * ROWS_PER_TILE, ROWS_PER_TILE)])


@functools.partial(
    pl.kernel,
    mesh=_mesh,
    out_type=jax.ShapeDtypeStruct((2, N_PAD, H_DIM), jnp.float32),
    scratch_types=[
        pltpu.VMEM((EW // 128, 128), jnp.int32),
        pltpu.VMEM((128, H_DIM), jnp.float32),
        pltpu.VMEM_SHARED((N_PAD, H_DIM), jnp.float32),
        pltpu.SemaphoreType.DMA,
    ],
    compiler_params=_sc_params,
)
def _deg_count(dst2d, out, dst_v, ones_v, acc_sh, sem):
    cid = lax.axis_index("c")
    sid = lax.axis_index("s")
    wid = cid * 16 + sid
    _zero_rows(ones_v, 128)
    def zslab(i, carry):
        pltpu.sync_copy(ones_v.at[pl.ds(0, 8)],
                        acc_sh.at[pl.ds(sid * ROWS_PER_TILE + i * 8, 8)])
        return carry
    lax.fori_loop(0, ROWS_PER_TILE // 8, zslab, 0)
    _fill_ones(ones_v, 128)
    pltpu.sync_copy(dst2d.at[pl.ds(wid * (EW // 128), EW // 128)], dst_v)
    plsc.subcore_barrier()

    pend = []
    for j in range(EW // 128):
        pend.append(pltpu.async_copy(ones_v, acc_sh.at[dst_v.at[j]], sem,
                                     add=True))
        if len(pend) > 16:
            pend.pop(0).wait()
    for c in pend:
        c.wait()
    plsc.subcore_barrier()
    pltpu.sync_copy(acc_sh.at[pl.ds(sid * ROWS_PER_TILE, ROWS_PER_TILE)],
                    out.at[cid, pl.ds(sid * ROWS_PER_TILE, ROWS_PER_TILE)])


_ROWS_BLK = 1000
_NBLK = N_NODES // _ROWS_BLK


def _tc_a_body(x_ref, w1_ref, degp_ref, s1_ref, norm_ref):
    cnt = degp_ref[0, :, 0:1] + degp_ref[1, :, 0:1] + 1.0
    nrm = lax.rsqrt(jnp.maximum(cnt, 1.0))
    hw = jnp.dot(x_ref[...], w1_ref[...], preferred_element_type=jnp.float32)
    s1_ref[...] = hw * nrm
    norm_ref[...] = nrm


def _tc_b_body(g1p_ref, s1_ref, norm_ref, b1_ref, s2_ref):
    nrm = norm_ref[...]
    h1 = jnp.maximum((g1p_ref[0] + g1p_ref[1] + s1_ref[...]) * nrm
                     + b1_ref[...], 0.0)
    s2_ref[...] = h1 * nrm


def _tc_c_body(g2p_ref, s2_ref, norm_ref, w2_ref, b2_ref, out_ref):
    agg = (g2p_ref[0] + g2p_ref[1] + s2_ref[...]) * norm_ref[...]
    z = jnp.dot(agg, w2_ref[...], preferred_element_type=jnp.float32) + b2_ref[...]
    m = jnp.max(z, axis=1, keepdims=True)
    lse = m + jnp.log(jnp.sum(jnp.exp(z - m), axis=1, keepdims=True))
    out_ref[...] = z - lse


_tc_a = pl.pallas_call(
    _tc_a_body,
    grid=(_NBLK,),
    in_specs=[
        pl.BlockSpec((_ROWS_BLK, F_IN), lambda i: (i, 0)),
        pl.BlockSpec((F_IN, H_DIM), lambda i: (0, 0)),
        pl.BlockSpec((2, _ROWS_BLK, H_DIM), lambda i: (0, i, 0)),
    ],
    out_specs=[
        pl.BlockSpec((_ROWS_BLK, H_DIM), lambda i: (i, 0)),
        pl.BlockSpec((_ROWS_BLK, 1), lambda i: (i, 0)),
    ],
    out_shape=[
        jax.ShapeDtypeStruct((N_NODES, H_DIM), jnp.float32),
        jax.ShapeDtypeStruct((N_NODES, 1), jnp.float32),
    ],
)

_tc_b = pl.pallas_call(
    _tc_b_body,
    grid=(_NBLK,),
    in_specs=[
        pl.BlockSpec((2, _ROWS_BLK, H_DIM), lambda i: (0, i, 0)),
        pl.BlockSpec((_ROWS_BLK, H_DIM), lambda i: (i, 0)),
        pl.BlockSpec((_ROWS_BLK, 1), lambda i: (i, 0)),
        pl.BlockSpec((1, H_DIM), lambda i: (0, 0)),
    ],
    out_specs=pl.BlockSpec((_ROWS_BLK, H_DIM), lambda i: (i, 0)),
    out_shape=jax.ShapeDtypeStruct((N_NODES, H_DIM), jnp.float32),
)

_tc_c = pl.pallas_call(
    _tc_c_body,
    grid=(_NBLK,),
    in_specs=[
        pl.BlockSpec((2, _ROWS_BLK, H_DIM), lambda i: (0, i, 0)),
        pl.BlockSpec((_ROWS_BLK, H_DIM), lambda i: (i, 0)),
        pl.BlockSpec((_ROWS_BLK, 1), lambda i: (i, 0)),
        pl.BlockSpec((H_DIM, C_DIM), lambda i: (0, 0)),
        pl.BlockSpec((1, C_DIM), lambda i: (0, 0)),
    ],
    out_specs=pl.BlockSpec((_ROWS_BLK, C_DIM), lambda i: (i, 0)),
    out_shape=jax.ShapeDtypeStruct((N_NODES, C_DIM), jnp.float32),
)


def kernel(x, edge_index, W1, b1, W2, b2):
    src = edge_index[0]
    dst = edge_index[1]
    pad = E_PAD - E_EDGES
    pad_src = jnp.arange(pad, dtype=jnp.int32) % N_NODES
    src2d = jnp.concatenate([src, pad_src]).reshape(E_PAD // 128, 128)
    pad_dst = N_NODES + jnp.arange(pad, dtype=jnp.int32) % (N_PAD - N_NODES)
    dst2d = jnp.concatenate([dst, pad_dst]).reshape(E_PAD // 128, 128)

    degp = _deg_count(dst2d)
    s1, norm = _tc_a(x, W1, degp)
    g1p = _seg_sum(s1, src2d, dst2d)
    s2 = _tc_b(g1p, s1, norm, b1.reshape(1, H_DIM))
    g2p = _seg_sum(s2, src2d, dst2d)
    return _tc_c(g2p, s2, norm, W2, b2.reshape(1, C_DIM))
